# baseline (speedup 1.0000x reference)
import math

import jax
import jax.numpy as jnp
from jax import lax
from jax.experimental import pallas as pl
from jax.experimental.pallas import tpu as pltpu
from jax.experimental.pallas import tpu_sc as plsc

V_ = 100000
B_ = 16384
NC_ = 2
NS_ = 16
L_ = 16
NW_ = NC_ * NS_
BPW_ = B_ // NW_
NIDX_ = BPW_ // 128
NCHUNK_ = BPW_ // L_

_CA = tuple(float((-1) ** k) / math.factorial(2 * k + 1) for k in range(9))
_CB = tuple(float((-1) ** k) / math.factorial(2 * k + 2) for k in range(9))


def _pose_body(ids_hbm, rx_hbm, ry_hbm, rz_hbm, tx_hbm, ty_hbm, tz_hbm,
               out_hbm, idx_v, k_v, comp_v, out_v, sem):
    wid = lax.axis_index("s") * NC_ + lax.axis_index("c")
    base = wid * BPW_

    for j in range(NIDX_):
        pltpu.sync_copy(ids_hbm.at[pl.ds(base + j * 128, 128)], idx_v.at[j])

    def mk_idx(i, carry):
        jrow = i // 8
        jcol = (i % 8) * L_
        k_v[jrow, pl.ds(jcol, L_)] = idx_v[jrow, pl.ds(jcol, L_)]
        return carry
    lax.fori_loop(0, NCHUNK_, mk_idx, 0, unroll=True)

    comps = (rx_hbm, ry_hbm, rz_hbm, tx_hbm, ty_hbm, tz_hbm)
    cps = []
    for c, src in enumerate(comps):
        for j in range(NIDX_):
            cps.append(pltpu.async_copy(
                src.at[k_v.at[j]],
                comp_v.at[c, pl.ds(j * 128, 128)], sem))
    for cp in cps:
        cp.wait()

    zeros = jnp.zeros((L_,), jnp.float32)
    ones = jnp.ones((L_,), jnp.float32)

    def chunk(i, carry):
        off = i * L_
        rows = off + lax.iota(jnp.int32, L_)
        x = comp_v[0, pl.ds(off, L_)]
        y = comp_v[1, pl.ds(off, L_)]
        z = comp_v[2, pl.ds(off, L_)]
        tx = comp_v[3, pl.ds(off, L_)]
        ty = comp_v[4, pl.ds(off, L_)]
        tz = comp_v[5, pl.ds(off, L_)]

        xx = x * x
        yy = y * y
        zz = z * z
        u = xx + yy + zz
        a = jnp.full((L_,), _CA[-1], jnp.float32)
        b = jnp.full((L_,), _CB[-1], jnp.float32)
        for k in range(len(_CA) - 2, -1, -1):
            a = a * u + jnp.float32(_CA[k])
            b = b * u + jnp.float32(_CB[k])
        xy = b * (x * y)
        xz = b * (x * z)
        yz = b * (y * z)
        ax = a * x
        ay = a * y
        az = a * z

        cols = (
            ones - b * (yy + zz), xy - az, ay + xz, tx,
            az + xy, ones - b * (xx + zz), yz - ax, ty,
            xz - ay, ax + yz, ones - b * (xx + yy), tz,
            zeros, zeros, zeros, ones,
        )
        for c, v in enumerate(cols):
            plsc.store_scatter(out_v, [rows, jnp.full((L_,), c, jnp.int32)], v)
        return carry

    lax.fori_loop(0, NCHUNK_, chunk, 0)
    pltpu.sync_copy(out_v.at[:, pl.ds(0, 16)], out_hbm.at[pl.ds(base, BPW_)])


@jax.jit
def kernel(cam_ids, r, t):
    mesh = plsc.VectorSubcoreMesh(
        core_axis_name="c", subcore_axis_name="s",
        num_cores=NC_, num_subcores=NS_)
    call = pl.kernel(
        _pose_body,
        out_type=jax.ShapeDtypeStruct((B_, 16), jnp.float32),
        mesh=mesh,
        scratch_types=[
            pltpu.VMEM((NIDX_, 128), jnp.int32),
            pltpu.VMEM((NIDX_, 128), jnp.int32),
            pltpu.VMEM((6, BPW_), jnp.float32),
            pltpu.VMEM((BPW_, 17), jnp.float32),
            pltpu.SemaphoreType.DMA,
        ],
        compiler_params=pltpu.CompilerParams(
            needs_layout_passes=False, use_tc_tiling_on_sc=False),
    )
    out = call(cam_ids.astype(jnp.int32),
               r[:, 0], r[:, 1], r[:, 2], t[:, 0], t[:, 1], t[:, 2])
    return out.reshape(B_, 4, 4)


# speedup vs baseline: 1.0483x; 1.0483x over previous
import math

import jax
import jax.numpy as jnp
from jax import lax
from jax.experimental import pallas as pl
from jax.experimental.pallas import tpu as pltpu
from jax.experimental.pallas import tpu_sc as plsc

V_ = 100000
B_ = 16384
NC_ = 2
NS_ = 16
L_ = 16
NW_ = NC_ * NS_
BPW_ = B_ // NW_
NIDX_ = BPW_ // 128
NCHUNK_ = BPW_ // L_

_CA = tuple(float((-1) ** k) / math.factorial(2 * k + 1) for k in range(9))
_CB = tuple(float((-1) ** k) / math.factorial(2 * k + 2) for k in range(9))


def _pose_body(ids_hbm, rx_hbm, ry_hbm, rz_hbm, tx_hbm, ty_hbm, tz_hbm,
               out_hbm, idx_v, k_v, comp_v, out_v, sem):
    wid = lax.axis_index("s") * NC_ + lax.axis_index("c")
    base = wid * BPW_

    for j in range(NIDX_):
        pltpu.sync_copy(ids_hbm.at[pl.ds(base + j * 128, 128)], idx_v.at[j])

    def mk_idx(i, carry):
        jrow = i // 8
        jcol = (i % 8) * L_
        k_v[jrow, pl.ds(jcol, L_)] = idx_v[jrow, pl.ds(jcol, L_)]
        return carry
    lax.fori_loop(0, NCHUNK_, mk_idx, 0, unroll=True)

    comps = (rx_hbm, ry_hbm, rz_hbm, tx_hbm, ty_hbm, tz_hbm)
    cps = []
    for c, src in enumerate(comps):
        for j in range(NIDX_):
            cps.append(pltpu.async_copy(
                src.at[k_v.at[j]],
                comp_v.at[c, pl.ds(j * 128, 128)], sem))
    for cp in cps:
        cp.wait()

    zeros = jnp.zeros((L_,), jnp.float32)
    ones = jnp.ones((L_,), jnp.float32)

    def chunk(i, carry):
        off = i * L_
        rows = off + lax.iota(jnp.int32, L_)
        x = comp_v[0, pl.ds(off, L_)]
        y = comp_v[1, pl.ds(off, L_)]
        z = comp_v[2, pl.ds(off, L_)]
        tx = comp_v[3, pl.ds(off, L_)]
        ty = comp_v[4, pl.ds(off, L_)]
        tz = comp_v[5, pl.ds(off, L_)]

        xx = x * x
        yy = y * y
        zz = z * z
        u = xx + yy + zz
        a = jnp.full((L_,), _CA[-1], jnp.float32)
        b = jnp.full((L_,), _CB[-1], jnp.float32)
        for k in range(len(_CA) - 2, -1, -1):
            a = a * u + jnp.float32(_CA[k])
            b = b * u + jnp.float32(_CB[k])
        xy = b * (x * y)
        xz = b * (x * z)
        yz = b * (y * z)
        ax = a * x
        ay = a * y
        az = a * z

        cols = (
            ones - b * (yy + zz), xy - az, ay + xz, tx,
            az + xy, ones - b * (xx + zz), yz - ax, ty,
            xz - ay, ax + yz, ones - b * (xx + yy), tz,
            zeros, zeros, zeros, ones,
        )
        for c, v in enumerate(cols):
            plsc.store_scatter(out_v, [rows, jnp.full((L_,), c, jnp.int32)], v)
        return carry

    lax.fori_loop(0, NCHUNK_, chunk, 0)
    pltpu.sync_copy(out_v, out_hbm.at[pl.ds(base, BPW_)])


@jax.jit
def kernel(cam_ids, r, t):
    mesh = plsc.VectorSubcoreMesh(
        core_axis_name="c", subcore_axis_name="s",
        num_cores=NC_, num_subcores=NS_)
    call = pl.kernel(
        _pose_body,
        out_type=jax.ShapeDtypeStruct((B_, 16), jnp.float32),
        mesh=mesh,
        scratch_types=[
            pltpu.VMEM((NIDX_, 128), jnp.int32),
            pltpu.VMEM((NIDX_, 128), jnp.int32),
            pltpu.VMEM((6, BPW_), jnp.float32),
            pltpu.VMEM((BPW_, 16), jnp.float32),
            pltpu.SemaphoreType.DMA,
        ],
        compiler_params=pltpu.CompilerParams(
            needs_layout_passes=False, use_tc_tiling_on_sc=False),
    )
    out = call(cam_ids.astype(jnp.int32),
               r[:, 0], r[:, 1], r[:, 2], t[:, 0], t[:, 1], t[:, 2])
    return out.reshape(B_, 4, 4)
